# baseline (speedup 1.0000x reference)
import functools

import jax
import jax.numpy as jnp
from jax import lax
from jax.experimental import pallas as pl
from jax.experimental.pallas import tpu as pltpu
from jax.experimental.pallas import tpu_sc as plsc

L_SEQ = 200
WORD_DIM = 64
POS_DIM = 64
HIDDEN = 128
LANES = 16
NC = 2
NS = 16
NCHUNK = 25

mesh = plsc.VectorSubcoreMesh(
    core_axis_name="c", subcore_axis_name="s", num_cores=NC, num_subcores=NS
)


def _m8(x):
    return pl.multiple_of(x, 8)


@functools.partial(
    pl.kernel,
    out_type=[
        jax.ShapeDtypeStruct((L_SEQ, HIDDEN), jnp.float32),
        jax.ShapeDtypeStruct((NC, HIDDEN), jnp.float32),
    ],
    mesh=mesh,
    compiler_params=pltpu.CompilerParams(
        needs_layout_passes=False,
        disable_bounds_checks=True,
        disable_semaphore_checks=True,
    ),
    scratch_types=[
        pltpu.VMEM((LANES,), jnp.int32),
        pltpu.VMEM((8, WORD_DIM, 128), jnp.float32),
        pltpu.VMEM((WORD_DIM, 128), jnp.float32),
        pltpu.VMEM((8, HIDDEN), jnp.float32),
        pltpu.VMEM((1, HIDDEN), jnp.float32),
        pltpu.VMEM((NS, HIDDEN), jnp.float32),
        pltpu.VMEM_SHARED((NS, HIDDEN), jnp.float32),
        pltpu.SemaphoreType.DMA,
        pltpu.SemaphoreType.DMA,
        pltpu.SemaphoreType.DMA,
    ],
)
def _encode(idx_hbm, wordt_hbm, post_hbm, out_hbm, hid_hbm,
            idx_v, blk_v, post_v, rows_v, psum_v, part_v, parts_s, sem,
            sem_idx, sem_pos):
    c = lax.axis_index("c")
    s = lax.axis_index("s")
    wid = s * NC + c

    zero = jnp.zeros((LANES,), jnp.float32)
    for cc in range(8):
        psum_v[0, pl.ds(cc * LANES, LANES)] = zero

    @pl.when(wid < NCHUNK)
    def _():
        base = _m8(wid * 8)
        cp_idx = pltpu.async_copy(
            idx_hbm.at[pl.ds(base, 8)], idx_v.at[pl.ds(0, 8)], sem_idx
        )
        pos_base = pl.multiple_of(jnp.where(wid < 16, 0, 128), 128)
        cp_pos = pltpu.async_copy(
            post_hbm.at[:, pl.ds(pos_base, 128)], post_v, sem_pos
        )
        cp_idx.wait()
        toks = idx_v[pl.ds(0, LANES)]
        rowids = [lax.iota(jnp.int32, LANES) + cc * LANES for cc in range(4)]

        cps = [
            pltpu.async_copy(
                wordt_hbm.at[
                    :, pl.ds(pl.multiple_of((toks[j] // 128) * 128, 128), 128)
                ],
                blk_v.at[j],
                sem,
            )
            for j in range(8)
        ]
        cp_pos.wait()
        for j in range(8):
            poscol = jnp.broadcast_to(base + j - pos_base, (LANES,)).astype(
                jnp.int32
            )
            for cc in range(4):
                p = plsc.load_gather(post_v, [rowids[cc], poscol])
                rows_v[j, pl.ds(WORD_DIM + cc * LANES, LANES)] = p

        for cp in cps:
            cp.wait()

        for j in range(8):
            lanecol = jnp.broadcast_to(toks[j] % 128, (LANES,)).astype(jnp.int32)
            for cc in range(4):
                g = plsc.load_gather(blk_v.at[j], [rowids[cc], lanecol])
                rows_v[j, pl.ds(cc * LANES, LANES)] = g

        cp_out = pltpu.async_copy(rows_v, out_hbm.at[pl.ds(base, 8)], sem)

        for cc in range(8):
            acc = rows_v[0, pl.ds(cc * LANES, LANES)]
            for j in range(1, 8):
                acc = acc + rows_v[j, pl.ds(cc * LANES, LANES)]
            psum_v[0, pl.ds(cc * LANES, LANES)] = acc
        cp_out.wait()

    pltpu.sync_copy(psum_v, parts_s.at[pl.ds(s, 1)])
    plsc.subcore_barrier()

    @pl.when(s == 0)
    def _():
        pltpu.sync_copy(parts_s, part_v)
        scale = jnp.float32(1.0 / L_SEQ)
        for cc in range(8):
            tot = part_v[0, pl.ds(cc * LANES, LANES)]
            for j in range(1, NS):
                tot = tot + part_v[j, pl.ds(cc * LANES, LANES)]
            psum_v[0, pl.ds(cc * LANES, LANES)] = tot * scale
        pltpu.sync_copy(psum_v, hid_hbm.at[pl.ds(c, 1)])


def kernel(inputs, W_word, W_pos):
    out, hid2 = _encode(inputs, W_word.T, W_pos.T)
    return out, (hid2[0] + hid2[1]).reshape(1, 1, HIDDEN)


# speedup vs baseline: 1.0021x; 1.0021x over previous
import functools

import jax
import jax.numpy as jnp
from jax import lax
from jax.experimental import pallas as pl
from jax.experimental.pallas import tpu as pltpu
from jax.experimental.pallas import tpu_sc as plsc

L_SEQ = 200
WORD_DIM = 64
POS_DIM = 64
HIDDEN = 128
LANES = 16
NC = 2
NS = 16
NCHUNK = 25

mesh = plsc.VectorSubcoreMesh(
    core_axis_name="c", subcore_axis_name="s", num_cores=NC, num_subcores=NS
)


def _m8(x):
    return pl.multiple_of(x, 8)


@functools.partial(
    pl.kernel,
    out_type=[
        jax.ShapeDtypeStruct((L_SEQ, HIDDEN), jnp.float32),
        jax.ShapeDtypeStruct((NC, HIDDEN), jnp.float32),
    ],
    mesh=mesh,
    compiler_params=pltpu.CompilerParams(
        needs_layout_passes=False,
        disable_bounds_checks=True,
        disable_semaphore_checks=True,
    ),
    scratch_types=[
        pltpu.VMEM((LANES,), jnp.int32),
        pltpu.VMEM((8, WORD_DIM, 128), jnp.float32),
        pltpu.VMEM((WORD_DIM, 128), jnp.float32),
        pltpu.VMEM((8, HIDDEN), jnp.float32),
        pltpu.VMEM((1, HIDDEN), jnp.float32),
        pltpu.VMEM((NS, HIDDEN), jnp.float32),
        pltpu.VMEM_SHARED((NS, HIDDEN), jnp.float32),
        pltpu.SemaphoreType.DMA,
        pltpu.SemaphoreType.DMA,
        pltpu.SemaphoreType.DMA,
    ],
)
def _encode(idx_hbm, wordt_hbm, post_hbm, out_hbm, hid_hbm,
            idx_v, blk_v, post_v, rows_v, psum_v, part_v, parts_s, sem,
            sem_idx, sem_pos):
    c = lax.axis_index("c")
    s = lax.axis_index("s")
    wid = s * NC + c

    zero = jnp.zeros((LANES,), jnp.float32)
    for cc in range(8):
        psum_v[0, pl.ds(cc * LANES, LANES)] = zero

    @pl.when(wid < NCHUNK)
    def _():
        base = _m8(wid * 8)
        cp_idx = pltpu.async_copy(
            idx_hbm.at[pl.ds(base, 8)], idx_v.at[pl.ds(0, 8)], sem_idx
        )
        pos_base = pl.multiple_of(jnp.where(wid < 16, 0, 128), 128)
        cp_pos = pltpu.async_copy(
            post_hbm.at[:, pl.ds(pos_base, 128)], post_v, sem_pos
        )
        cp_idx.wait()
        toks = idx_v[pl.ds(0, LANES)]
        rowids = [lax.iota(jnp.int32, LANES) + cc * LANES for cc in range(4)]

        cps = [
            pltpu.async_copy(
                wordt_hbm.at[
                    :, pl.ds(pl.multiple_of((toks[j] // 128) * 128, 128), 128)
                ],
                blk_v.at[j],
                sem,
            )
            for j in range(8)
        ]
        cp_pos.wait()
        accp = [jnp.zeros((LANES,), jnp.float32) for _ in range(4)]
        for j in range(8):
            poscol = jnp.broadcast_to(base + j - pos_base, (LANES,)).astype(
                jnp.int32
            )
            for cc in range(4):
                p = plsc.load_gather(post_v, [rowids[cc], poscol])
                rows_v[j, pl.ds(WORD_DIM + cc * LANES, LANES)] = p
                accp[cc] = accp[cc] + p

        for cp in cps:
            cp.wait()

        accw = [jnp.zeros((LANES,), jnp.float32) for _ in range(4)]
        for j in range(8):
            lanecol = jnp.broadcast_to(toks[j] % 128, (LANES,)).astype(jnp.int32)
            for cc in range(4):
                g = plsc.load_gather(blk_v.at[j], [rowids[cc], lanecol])
                rows_v[j, pl.ds(cc * LANES, LANES)] = g
                accw[cc] = accw[cc] + g

        cp_out = pltpu.async_copy(rows_v, out_hbm.at[pl.ds(base, 8)], sem)
        for cc in range(4):
            psum_v[0, pl.ds(cc * LANES, LANES)] = accw[cc]
            psum_v[0, pl.ds(WORD_DIM + cc * LANES, LANES)] = accp[cc]
        cp_out.wait()

    pltpu.sync_copy(psum_v, parts_s.at[pl.ds(s, 1)])
    plsc.subcore_barrier()

    @pl.when(s == 0)
    def _():
        pltpu.sync_copy(parts_s, part_v)
        scale = jnp.float32(1.0 / L_SEQ)
        for cc in range(8):
            tot = part_v[0, pl.ds(cc * LANES, LANES)]
            for j in range(1, NS):
                tot = tot + part_v[j, pl.ds(cc * LANES, LANES)]
            psum_v[0, pl.ds(cc * LANES, LANES)] = tot * scale
        pltpu.sync_copy(psum_v, hid_hbm.at[pl.ds(c, 1)])


def kernel(inputs, W_word, W_pos):
    out, hid2 = _encode(inputs, W_word.T, W_pos.T)
    return out, (hid2[0] + hid2[1]).reshape(1, 1, HIDDEN)
